# baseline (speedup 1.0000x reference)
import functools

import jax
import jax.numpy as jnp
from jax import lax
from jax.experimental import pallas as pl
from jax.experimental.pallas import tpu as pltpu
from jax.experimental.pallas import tpu_sc as plsc

N_V = 400002
N_D = 200
D_PAD = 256

NC = 2
NS = 16
NW = NC * NS

CHUNK = 128


def _gather_body(idx_hbm, table_hbm, out_hbm, idx_v, buf, sem):
    c = lax.axis_index("c")
    s = lax.axis_index("s")
    wid = s * NC + c
    n_chunks = idx_v.shape[0]
    pltpu.sync_copy(idx_hbm.at[wid], idx_v)
    base = wid * (n_chunks * CHUNK)

    def step(j, carry):
        pltpu.async_copy(table_hbm.at[idx_v.at[j]], buf, sem).wait()
        pltpu.sync_copy(buf, out_hbm.at[pl.ds(base + j * CHUNK, CHUNK)])
        return carry

    lax.fori_loop(0, n_chunks, step, 0)


@functools.partial(jax.jit, static_argnames=("n_chunks",))
def _gather(idx3, table_pad, n_chunks):
    mesh = plsc.VectorSubcoreMesh(
        core_axis_name="c", subcore_axis_name="s", num_cores=NC, num_subcores=NS
    )
    total = NW * n_chunks * CHUNK
    return pl.kernel(
        _gather_body,
        out_type=jax.ShapeDtypeStruct((total, D_PAD), jnp.float32),
        mesh=mesh,
        scratch_types=[
            pltpu.VMEM((n_chunks, CHUNK), jnp.int32),
            pltpu.VMEM((CHUNK, D_PAD), jnp.float32),
            pltpu.SemaphoreType.DMA,
        ],
    )(idx3, table_pad)


def kernel(input, weight):
    B, S = input.shape
    total = B * S
    assert total % (NW * CHUNK) == 0
    n_chunks = total // (NW * CHUNK)
    idx3 = input.reshape(NW, n_chunks, CHUNK).astype(jnp.int32)
    table_pad = jnp.pad(weight, ((0, 0), (0, D_PAD - N_D)))
    out = _gather(idx3, table_pad, n_chunks)
    return out[:, :N_D].reshape(B, S, N_D)


# speedup vs baseline: 1.0142x; 1.0142x over previous
import functools

import jax
import jax.numpy as jnp
from jax import lax
from jax.experimental import pallas as pl
from jax.experimental.pallas import tpu as pltpu
from jax.experimental.pallas import tpu_sc as plsc

N_V = 400002
N_D = 200
D_PAD = 256
TAIL = N_D - 128

NC = 2
NS = 16
NW = NC * NS

CHUNK = 128


def _gather_body(idx_hbm, table_hbm, out_hbm,
                 idx_v, buf_a, buf_b, buf_t,
                 sem_a0, sem_a1, sem_b0, sem_b1):
    c = lax.axis_index("c")
    s = lax.axis_index("s")
    wid = s * NC + c
    n_chunks = idx_v.shape[0]
    pltpu.sync_copy(idx_hbm.at[wid], idx_v)
    base = wid * (n_chunks * CHUNK)
    sems_a = (sem_a0, sem_a1)
    sems_b = (sem_b0, sem_b1)

    def start(j, slot):
        pltpu.async_copy(
            table_hbm.at[idx_v.at[j], pl.ds(0, 128)], buf_a.at[slot],
            sems_a[slot])
        pltpu.async_copy(
            table_hbm.at[idx_v.at[j], pl.ds(128, 128)], buf_b.at[slot],
            sems_b[slot])

    def wait(slot):
        pltpu.make_async_copy(
            table_hbm.at[idx_v.at[0], pl.ds(0, 128)], buf_a.at[slot],
            sems_a[slot]).wait()
        pltpu.make_async_copy(
            table_hbm.at[idx_v.at[0], pl.ds(128, 128)], buf_b.at[slot],
            sems_b[slot]).wait()

    def repack(slot):
        bb = buf_b.at[slot]

        def row(i, carry):
            for k in range(4):
                buf_t[i, pl.ds(16 * k, 16)] = bb[i, pl.ds(16 * k, 16)]
            buf_t[i, pl.ds(TAIL - 16, 16)] = bb[i, pl.ds(TAIL - 16, 16)]
            return carry

        lax.fori_loop(0, CHUNK, row, 0)

    def flush(j, slot):
        rows = pl.ds(base + j * CHUNK, CHUNK)
        pltpu.sync_copy(buf_a.at[slot], out_hbm.at[rows, pl.ds(0, 128)])
        repack(slot)
        pltpu.sync_copy(buf_t, out_hbm.at[rows, pl.ds(128, TAIL)])

    start(0, 0)

    def group(g, carry):
        for b in range(2):
            j = 2 * g + b
            wait(b)

            @pl.when(j + 1 < n_chunks)
            def _():
                start(j + 1, 1 - b)

            flush(j, b)
        return carry

    lax.fori_loop(0, n_chunks // 2, group, 0)


@functools.partial(jax.jit, static_argnames=("n_chunks",))
def _gather(idx3, table_pad, n_chunks):
    mesh = plsc.VectorSubcoreMesh(
        core_axis_name="c", subcore_axis_name="s", num_cores=NC, num_subcores=NS
    )
    total = NW * n_chunks * CHUNK
    return pl.kernel(
        _gather_body,
        out_type=jax.ShapeDtypeStruct((total, N_D), jnp.float32),
        mesh=mesh,
        scratch_types=[
            pltpu.VMEM((n_chunks, CHUNK), jnp.int32),
            pltpu.VMEM((2, CHUNK, 128), jnp.float32),
            pltpu.VMEM((2, CHUNK, 128), jnp.float32),
            pltpu.VMEM((CHUNK, TAIL), jnp.float32),
            pltpu.SemaphoreType.DMA,
            pltpu.SemaphoreType.DMA,
            pltpu.SemaphoreType.DMA,
            pltpu.SemaphoreType.DMA,
        ],
    )(idx3, table_pad)


def kernel(input, weight):
    B, S = input.shape
    total = B * S
    assert total % (NW * CHUNK) == 0
    n_chunks = total // (NW * CHUNK)
    assert n_chunks % 2 == 0
    idx3 = input.reshape(NW, n_chunks, CHUNK).astype(jnp.int32)
    table_pad = jnp.pad(weight, ((0, 0), (0, D_PAD - N_D)))
    out = _gather(idx3, table_pad, n_chunks)
    return out.reshape(B, S, N_D)


# speedup vs baseline: 2.6041x; 2.5676x over previous
import functools

import jax
import jax.numpy as jnp
from jax import lax
from jax.experimental import pallas as pl
from jax.experimental.pallas import tpu as pltpu
from jax.experimental.pallas import tpu_sc as plsc

N_V = 400002
N_D = 200
D_PAD = 256
TAIL = N_D - 128

NC = 2
NS = 16
NW = NC * NS

CHUNK = 128

BLK_V = 1024


def _transpose_body(wt_ref, eye_ref, out_ref):
    x = wt_ref[...]
    r = lax.dot_general(x, eye_ref[...], (((0,), (0,)), ((), ())),
                        preferred_element_type=jnp.float32)
    out_ref[:, pl.ds(0, N_D)] = r


def _transpose_tc(wt, eye):
    grid = -(-N_V // BLK_V)
    return pl.pallas_call(
        _transpose_body,
        grid=(grid,),
        in_specs=[
            pl.BlockSpec((N_D, BLK_V), lambda j: (0, j)),
            pl.BlockSpec((N_D, N_D), lambda j: (0, 0)),
        ],
        out_specs=pl.BlockSpec((BLK_V, D_PAD), lambda j: (j, 0)),
        out_shape=jax.ShapeDtypeStruct((N_V, D_PAD), jnp.float32),
    )(wt, eye)


def _gather_body(idx_hbm, table_hbm, out_hbm,
                 idx_v, buf_a, buf_b, buf_t,
                 sem_a0, sem_a1, sem_b0, sem_b1):
    c = lax.axis_index("c")
    s = lax.axis_index("s")
    wid = s * NC + c
    n_chunks = idx_v.shape[0]
    pltpu.sync_copy(idx_hbm.at[wid], idx_v)
    base = wid * (n_chunks * CHUNK)
    sems_a = (sem_a0, sem_a1)
    sems_b = (sem_b0, sem_b1)

    def start(j, slot):
        pltpu.async_copy(
            table_hbm.at[idx_v.at[j], pl.ds(0, 128)], buf_a.at[slot],
            sems_a[slot])
        pltpu.async_copy(
            table_hbm.at[idx_v.at[j], pl.ds(128, 128)], buf_b.at[slot],
            sems_b[slot])

    def wait(slot):
        pltpu.make_async_copy(
            table_hbm.at[idx_v.at[0], pl.ds(0, 128)], buf_a.at[slot],
            sems_a[slot]).wait()
        pltpu.make_async_copy(
            table_hbm.at[idx_v.at[0], pl.ds(128, 128)], buf_b.at[slot],
            sems_b[slot]).wait()

    def repack(slot):
        bb = buf_b.at[slot]

        def row(i, carry):
            for k in range(4):
                buf_t[i, pl.ds(16 * k, 16)] = bb[i, pl.ds(16 * k, 16)]
            buf_t[i, pl.ds(TAIL - 16, 16)] = bb[i, pl.ds(TAIL - 16, 16)]
            return carry

        lax.fori_loop(0, CHUNK, row, 0)

    def flush(j, slot):
        rows = pl.ds(base + j * CHUNK, CHUNK)
        pltpu.sync_copy(buf_a.at[slot], out_hbm.at[rows, pl.ds(0, 128)])
        repack(slot)
        pltpu.sync_copy(buf_t, out_hbm.at[rows, pl.ds(128, TAIL)])

    start(0, 0)

    def group(g, carry):
        for b in range(2):
            j = 2 * g + b
            wait(b)

            @pl.when(j + 1 < n_chunks)
            def _():
                start(j + 1, 1 - b)

            flush(j, b)
        return carry

    lax.fori_loop(0, n_chunks // 2, group, 0)


@functools.partial(jax.jit, static_argnames=("n_chunks",))
def _run(idx3, wt, eye, n_chunks):
    t2 = _transpose_tc(wt, eye)
    mesh = plsc.VectorSubcoreMesh(
        core_axis_name="c", subcore_axis_name="s", num_cores=NC, num_subcores=NS
    )
    total = NW * n_chunks * CHUNK
    return pl.kernel(
        _gather_body,
        out_type=jax.ShapeDtypeStruct((total, N_D), jnp.float32),
        mesh=mesh,
        scratch_types=[
            pltpu.VMEM((n_chunks, CHUNK), jnp.int32),
            pltpu.VMEM((2, CHUNK, 128), jnp.float32),
            pltpu.VMEM((2, CHUNK, 128), jnp.float32),
            pltpu.VMEM((CHUNK, TAIL), jnp.float32),
            pltpu.SemaphoreType.DMA,
            pltpu.SemaphoreType.DMA,
            pltpu.SemaphoreType.DMA,
            pltpu.SemaphoreType.DMA,
        ],
    )(idx3, t2)


def kernel(input, weight):
    B, S = input.shape
    total = B * S
    assert total % (NW * CHUNK) == 0
    n_chunks = total // (NW * CHUNK)
    assert n_chunks % 2 == 0
    idx3 = input.reshape(NW, n_chunks, CHUNK).astype(jnp.int32)
    wt = weight.T
    eye = jnp.eye(N_D, dtype=jnp.float32)
    out = _run(idx3, wt, eye, n_chunks)
    return out.reshape(B, S, N_D)


# speedup vs baseline: 3.0018x; 1.1527x over previous
import functools

import jax
import jax.numpy as jnp
from jax import lax
from jax.experimental import pallas as pl
from jax.experimental.pallas import tpu as pltpu
from jax.experimental.pallas import tpu_sc as plsc

N_V = 400002
N_D = 200
D_PAD = 256
TAIL = N_D - 128

NC = 2
NS = 16
NW = NC * NS

CHUNK = 128

BLK_V = 2048


def _transpose_body(wt_ref, eye_ref, out_ref):
    x = wt_ref[...]
    r = lax.dot_general(x, eye_ref[...], (((0,), (0,)), ((), ())),
                        preferred_element_type=jnp.float32)
    out_ref[:, pl.ds(0, N_D)] = r


def _transpose_tc(wt, eye):
    grid = -(-N_V // BLK_V)
    return pl.pallas_call(
        _transpose_body,
        grid=(grid,),
        in_specs=[
            pl.BlockSpec((N_D, BLK_V), lambda j: (0, j)),
            pl.BlockSpec((N_D, N_D), lambda j: (0, 0)),
        ],
        out_specs=pl.BlockSpec((BLK_V, D_PAD), lambda j: (j, 0)),
        out_shape=jax.ShapeDtypeStruct((N_V, D_PAD), jnp.float32),
    )(wt, eye)


def _gather_body(idx_hbm, table_hbm, out_hbm,
                 idx_v, buf_a, buf_b, buf_t,
                 sem_a0, sem_a1, sem_b0, sem_b1):
    c = lax.axis_index("c")
    s = lax.axis_index("s")
    wid = s * NC + c
    n_chunks = idx_v.shape[0]
    pltpu.sync_copy(idx_hbm.at[wid], idx_v)
    base = wid * (n_chunks * CHUNK)
    sems_a = (sem_a0, sem_a1)
    sems_b = (sem_b0, sem_b1)

    def start(j, slot):
        pltpu.async_copy(
            table_hbm.at[idx_v.at[j], pl.ds(0, 128)], buf_a.at[slot],
            sems_a[slot])
        pltpu.async_copy(
            table_hbm.at[idx_v.at[j], pl.ds(128, 128)], buf_b.at[slot],
            sems_b[slot])

    def wait(slot):
        pltpu.make_async_copy(
            table_hbm.at[idx_v.at[0], pl.ds(0, 128)], buf_a.at[slot],
            sems_a[slot]).wait()
        pltpu.make_async_copy(
            table_hbm.at[idx_v.at[0], pl.ds(128, 128)], buf_b.at[slot],
            sems_b[slot]).wait()

    def repack(slot):
        bb = buf_b.at[slot]

        def row(i, carry):
            for k in range(4):
                buf_t[i, pl.ds(16 * k, 16)] = bb[i, pl.ds(16 * k, 16)]
            buf_t[i, pl.ds(TAIL - 16, 16)] = bb[i, pl.ds(TAIL - 16, 16)]
            return carry

        lax.fori_loop(0, CHUNK, row, 0)

    def flush(j, slot):
        rows = pl.ds(base + j * CHUNK, CHUNK)
        pltpu.sync_copy(buf_a.at[slot], out_hbm.at[rows, pl.ds(0, 128)])
        repack(slot)
        pltpu.sync_copy(buf_t, out_hbm.at[rows, pl.ds(128, TAIL)])

    start(0, 0)

    def group(g, carry):
        for b in range(2):
            j = 2 * g + b
            wait(b)

            @pl.when(j + 1 < n_chunks)
            def _():
                start(j + 1, 1 - b)

            flush(j, b)
        return carry

    lax.fori_loop(0, n_chunks // 2, group, 0)


@functools.partial(jax.jit, static_argnames=("n_chunks",))
def _run(idx3, wt, eye, n_chunks):
    t2 = _transpose_tc(wt, eye)
    mesh = plsc.VectorSubcoreMesh(
        core_axis_name="c", subcore_axis_name="s", num_cores=NC, num_subcores=NS
    )
    total = NW * n_chunks * CHUNK
    return pl.kernel(
        _gather_body,
        out_type=jax.ShapeDtypeStruct((total, N_D), jnp.float32),
        mesh=mesh,
        scratch_types=[
            pltpu.VMEM((n_chunks, CHUNK), jnp.int32),
            pltpu.VMEM((2, CHUNK, 128), jnp.float32),
            pltpu.VMEM((2, CHUNK, 128), jnp.float32),
            pltpu.VMEM((CHUNK, TAIL), jnp.float32),
            pltpu.SemaphoreType.DMA,
            pltpu.SemaphoreType.DMA,
            pltpu.SemaphoreType.DMA,
            pltpu.SemaphoreType.DMA,
        ],
    )(idx3, t2)


def kernel(input, weight):
    B, S = input.shape
    total = B * S
    assert total % (NW * CHUNK) == 0
    n_chunks = total // (NW * CHUNK)
    assert n_chunks % 2 == 0
    idx3 = input.reshape(NW, n_chunks, CHUNK).astype(jnp.int32)
    wt = weight.T
    eye = jnp.eye(N_D, dtype=jnp.float32)
    out = _run(idx3, wt, eye, n_chunks)
    return out.reshape(B, S, N_D)


# speedup vs baseline: 3.2270x; 1.0750x over previous
import functools

import jax
import jax.numpy as jnp
from jax import lax
from jax.experimental import pallas as pl
from jax.experimental.pallas import tpu as pltpu
from jax.experimental.pallas import tpu_sc as plsc

N_V = 400002
N_D = 200
D_PAD = 256
TAIL = N_D - 128

NC = 2
NS = 16
NW = NC * NS

CHUNK = 128

BLK_V = 4096


def _transpose_body(wt_ref, eye_ref, out_ref):
    x = wt_ref[...]
    r = lax.dot_general(x, eye_ref[...], (((0,), (0,)), ((), ())),
                        preferred_element_type=jnp.float32)
    out_ref[:, pl.ds(0, N_D)] = r


def _transpose_tc(wt, eye):
    grid = -(-N_V // BLK_V)
    return pl.pallas_call(
        _transpose_body,
        grid=(grid,),
        in_specs=[
            pl.BlockSpec((N_D, BLK_V), lambda j: (0, j)),
            pl.BlockSpec((N_D, N_D), lambda j: (0, 0)),
        ],
        out_specs=pl.BlockSpec((BLK_V, D_PAD), lambda j: (j, 0)),
        out_shape=jax.ShapeDtypeStruct((N_V, D_PAD), jnp.float32),
    )(wt, eye)


def _gather_body(idx_hbm, table_hbm, out_hbm,
                 idx_v, buf_a, buf_b, buf_t,
                 sem_a0, sem_a1, sem_b0, sem_b1):
    c = lax.axis_index("c")
    s = lax.axis_index("s")
    wid = s * NC + c
    n_chunks = idx_v.shape[0]
    pltpu.sync_copy(idx_hbm.at[wid], idx_v)
    base = wid * (n_chunks * CHUNK)
    sems_a = (sem_a0, sem_a1)
    sems_b = (sem_b0, sem_b1)

    def start(j, slot):
        pltpu.async_copy(
            table_hbm.at[idx_v.at[j], pl.ds(0, 128)], buf_a.at[slot],
            sems_a[slot])
        pltpu.async_copy(
            table_hbm.at[idx_v.at[j], pl.ds(128, 128)], buf_b.at[slot],
            sems_b[slot])

    def wait(slot):
        pltpu.make_async_copy(
            table_hbm.at[idx_v.at[0], pl.ds(0, 128)], buf_a.at[slot],
            sems_a[slot]).wait()
        pltpu.make_async_copy(
            table_hbm.at[idx_v.at[0], pl.ds(128, 128)], buf_b.at[slot],
            sems_b[slot]).wait()

    def repack(slot):
        bb = buf_b.at[slot]

        def row(i, carry):
            for k in range(4):
                buf_t[i, pl.ds(16 * k, 16)] = bb[i, pl.ds(16 * k, 16)]
            buf_t[i, pl.ds(TAIL - 16, 16)] = bb[i, pl.ds(TAIL - 16, 16)]
            return carry

        lax.fori_loop(0, CHUNK, row, 0)

    def flush(j, slot):
        rows = pl.ds(base + j * CHUNK, CHUNK)
        pltpu.sync_copy(buf_a.at[slot], out_hbm.at[rows, pl.ds(0, 128)])
        repack(slot)
        pltpu.sync_copy(buf_t, out_hbm.at[rows, pl.ds(128, TAIL)])

    start(0, 0)

    def group(g, carry):
        for b in range(2):
            j = 2 * g + b
            wait(b)

            @pl.when(j + 1 < n_chunks)
            def _():
                start(j + 1, 1 - b)

            flush(j, b)
        return carry

    lax.fori_loop(0, n_chunks // 2, group, 0)


@functools.partial(jax.jit, static_argnames=("n_chunks",))
def _run(idx3, wt, eye, n_chunks):
    t2 = _transpose_tc(wt, eye)
    mesh = plsc.VectorSubcoreMesh(
        core_axis_name="c", subcore_axis_name="s", num_cores=NC, num_subcores=NS
    )
    total = NW * n_chunks * CHUNK
    return pl.kernel(
        _gather_body,
        out_type=jax.ShapeDtypeStruct((total, N_D), jnp.float32),
        mesh=mesh,
        scratch_types=[
            pltpu.VMEM((n_chunks, CHUNK), jnp.int32),
            pltpu.VMEM((2, CHUNK, 128), jnp.float32),
            pltpu.VMEM((2, CHUNK, 128), jnp.float32),
            pltpu.VMEM((CHUNK, TAIL), jnp.float32),
            pltpu.SemaphoreType.DMA,
            pltpu.SemaphoreType.DMA,
            pltpu.SemaphoreType.DMA,
            pltpu.SemaphoreType.DMA,
        ],
    )(idx3, t2)


def kernel(input, weight):
    B, S = input.shape
    total = B * S
    assert total % (NW * CHUNK) == 0
    n_chunks = total // (NW * CHUNK)
    assert n_chunks % 2 == 0
    idx3 = input.reshape(NW, n_chunks, CHUNK).astype(jnp.int32)
    wt = weight.T
    eye = jnp.eye(N_D, dtype=jnp.float32)
    out = _run(idx3, wt, eye, n_chunks)
    return out.reshape(B, S, N_D)


# speedup vs baseline: 3.2609x; 1.0105x over previous
import functools

import jax
import jax.numpy as jnp
from jax import lax
from jax.experimental import pallas as pl
from jax.experimental.pallas import tpu as pltpu
from jax.experimental.pallas import tpu_sc as plsc

N_V = 400002
N_D = 200
D_PAD = 256
TAIL = N_D - 128

NC = 2
NS = 16
NW = NC * NS

CHUNK = 128

BLK_V = 8192


def _transpose_body(wt_ref, eye_ref, out_ref):
    x = wt_ref[...]
    r = lax.dot_general(x, eye_ref[...], (((0,), (0,)), ((), ())),
                        preferred_element_type=jnp.float32)
    out_ref[:, pl.ds(0, N_D)] = r


def _transpose_tc(wt, eye):
    grid = -(-N_V // BLK_V)
    return pl.pallas_call(
        _transpose_body,
        grid=(grid,),
        in_specs=[
            pl.BlockSpec((N_D, BLK_V), lambda j: (0, j)),
            pl.BlockSpec((N_D, N_D), lambda j: (0, 0)),
        ],
        out_specs=pl.BlockSpec((BLK_V, D_PAD), lambda j: (j, 0)),
        out_shape=jax.ShapeDtypeStruct((N_V, D_PAD), jnp.float32),
    )(wt, eye)


def _gather_body(idx_hbm, table_hbm, out_hbm,
                 idx_v, buf_a, buf_b, buf_t,
                 sem_a0, sem_a1, sem_b0, sem_b1):
    c = lax.axis_index("c")
    s = lax.axis_index("s")
    wid = s * NC + c
    n_chunks = idx_v.shape[0]
    pltpu.sync_copy(idx_hbm.at[wid], idx_v)
    base = wid * (n_chunks * CHUNK)
    sems_a = (sem_a0, sem_a1)
    sems_b = (sem_b0, sem_b1)

    def start(j, slot):
        pltpu.async_copy(
            table_hbm.at[idx_v.at[j], pl.ds(0, 128)], buf_a.at[slot],
            sems_a[slot])
        pltpu.async_copy(
            table_hbm.at[idx_v.at[j], pl.ds(128, 128)], buf_b.at[slot],
            sems_b[slot])

    def wait(slot):
        pltpu.make_async_copy(
            table_hbm.at[idx_v.at[0], pl.ds(0, 128)], buf_a.at[slot],
            sems_a[slot]).wait()
        pltpu.make_async_copy(
            table_hbm.at[idx_v.at[0], pl.ds(128, 128)], buf_b.at[slot],
            sems_b[slot]).wait()

    def repack(slot):
        bb = buf_b.at[slot]

        def row(i, carry):
            for k in range(4):
                buf_t[i, pl.ds(16 * k, 16)] = bb[i, pl.ds(16 * k, 16)]
            buf_t[i, pl.ds(TAIL - 16, 16)] = bb[i, pl.ds(TAIL - 16, 16)]
            return carry

        lax.fori_loop(0, CHUNK, row, 0)

    def flush(j, slot):
        rows = pl.ds(base + j * CHUNK, CHUNK)
        pltpu.sync_copy(buf_a.at[slot], out_hbm.at[rows, pl.ds(0, 128)])
        repack(slot)
        pltpu.sync_copy(buf_t, out_hbm.at[rows, pl.ds(128, TAIL)])

    start(0, 0)

    def group(g, carry):
        for b in range(2):
            j = 2 * g + b
            wait(b)

            @pl.when(j + 1 < n_chunks)
            def _():
                start(j + 1, 1 - b)

            flush(j, b)
        return carry

    lax.fori_loop(0, n_chunks // 2, group, 0)


@functools.partial(jax.jit, static_argnames=("n_chunks",))
def _run(idx3, wt, eye, n_chunks):
    t2 = _transpose_tc(wt, eye)
    mesh = plsc.VectorSubcoreMesh(
        core_axis_name="c", subcore_axis_name="s", num_cores=NC, num_subcores=NS
    )
    total = NW * n_chunks * CHUNK
    return pl.kernel(
        _gather_body,
        out_type=jax.ShapeDtypeStruct((total, N_D), jnp.float32),
        mesh=mesh,
        scratch_types=[
            pltpu.VMEM((n_chunks, CHUNK), jnp.int32),
            pltpu.VMEM((2, CHUNK, 128), jnp.float32),
            pltpu.VMEM((2, CHUNK, 128), jnp.float32),
            pltpu.VMEM((CHUNK, TAIL), jnp.float32),
            pltpu.SemaphoreType.DMA,
            pltpu.SemaphoreType.DMA,
            pltpu.SemaphoreType.DMA,
            pltpu.SemaphoreType.DMA,
        ],
    )(idx3, t2)


def kernel(input, weight):
    B, S = input.shape
    total = B * S
    assert total % (NW * CHUNK) == 0
    n_chunks = total // (NW * CHUNK)
    assert n_chunks % 2 == 0
    idx3 = input.reshape(NW, n_chunks, CHUNK).astype(jnp.int32)
    wt = weight.T
    eye = jnp.eye(N_D, dtype=jnp.float32)
    out = _run(idx3, wt, eye, n_chunks)
    return out.reshape(B, S, N_D)
